# baseline (speedup 1.0000x reference)
import functools

import jax
import jax.numpy as jnp
from jax import lax
from jax.experimental import pallas as pl
from jax.experimental.pallas import tpu as pltpu
from jax.experimental.pallas import tpu_sc as plsc

N = 10000
E = 320000
D = 128
H = 128
C = 64

NC = 2
NS = 16
NW = NC * NS

CHUNK = 128
NCHUNKS = 79
EPW = NCHUNKS * CHUNK
E_PAD = NW * EPW
NPAD = 10240
RPT = NPAD // NS
RBLK = RPT // CHUNK
NROW = NPAD // CHUNK

_MESH = plsc.VectorSubcoreMesh(core_axis_name="c", subcore_axis_name="s")


@functools.partial(
    pl.kernel,
    out_type=jax.ShapeDtypeStruct((NC, NROW, CHUNK), jnp.float32),
    mesh=_MESH,
    scratch_types=[
        pltpu.VMEM((NCHUNKS, CHUNK), jnp.int32),
        pltpu.VMEM((NROW, CHUNK), jnp.float32),
        pltpu.VMEM((NROW // NS, CHUNK), jnp.float32),
        pltpu.VMEM((NROW,), jnp.int32),
        pltpu.VMEM_SHARED((NROW, CHUNK), jnp.float32),
    ],
    compiler_params=pltpu.CompilerParams(use_tc_tiling_on_sc=False,
                                         needs_layout_passes=False),
)
def _sc_degree(col_hbm, out_hbm, colbuf, hist, bounce, idbuf, acc):
    cid = lax.axis_index("c")
    sid = lax.axis_index("s")
    wid = sid * NC + cid
    pltpu.sync_copy(col_hbm.at[wid], colbuf)

    zero16 = jnp.zeros((16,), jnp.float32)
    one16 = jnp.ones((16,), jnp.float32)
    for r in range(NROW // NS):
        for j in range(CHUNK // 16):
            bounce[r, pl.ds(j * 16, 16)] = zero16
    for k in range(NROW // 16):
        idbuf[pl.ds(k * 16, 16)] = lax.iota(jnp.int32, 16) + (k * 16)

    def zrow(r, carry):
        for j in range(CHUNK // 16):
            hist[r, pl.ds(j * 16, 16)] = zero16
        return carry

    lax.fori_loop(0, NROW, zrow, 0)
    pltpu.sync_copy(bounce, acc.at[pl.ds(sid * (NROW // NS), NROW // NS)])

    def body(c, carry):
        for j in range(CHUNK // 16):
            idx = colbuf[c, pl.ds(j * 16, 16)]
            plsc.addupdate_scatter(
                hist, [lax.shift_right_logical(idx, 7),
                       lax.bitwise_and(idx, 127)], one16)
        return carry

    lax.fori_loop(0, NCHUNKS, body, 0)
    plsc.subcore_barrier()
    pltpu.sync_copy(hist, acc.at[idbuf], add=True)
    plsc.subcore_barrier()
    pltpu.sync_copy(acc.at[pl.ds(sid * (NROW // NS), NROW // NS)], bounce)
    pltpu.sync_copy(bounce,
                    out_hbm.at[cid, pl.ds(sid * (NROW // NS), NROW // NS)])


def _make_sc_agg(width, chunk, nch0, nch1):
    nmax = max(nch0, nch1)
    rblk = RPT // chunk

    @functools.partial(
        pl.kernel,
        out_type=jax.ShapeDtypeStruct((NC, NPAD, width), jnp.bfloat16),
        mesh=_MESH,
        scratch_types=[
            pltpu.VMEM((nmax, chunk), jnp.int32),
            pltpu.VMEM((nmax, chunk), jnp.int32),
            pltpu.VMEM((chunk, width), jnp.bfloat16),
            pltpu.VMEM((chunk, width), jnp.bfloat16),
            pltpu.VMEM_SHARED((NPAD, width), jnp.bfloat16),
            pltpu.SemaphoreType.DMA,
            pltpu.SemaphoreType.DMA,
        ],
        compiler_params=pltpu.CompilerParams(use_tc_tiling_on_sc=False),
    )
    def agg(g_hbm, row0_hbm, col0_hbm, row1_hbm, col1_hbm, out_hbm,
            rowbuf, colbuf, rows_a, rows_b, acc, sem_a, sem_b):
        cid = lax.axis_index("c")
        sid = lax.axis_index("s")

        @pl.when(cid == 0)
        def _():
            pltpu.sync_copy(row0_hbm.at[sid], rowbuf.at[pl.ds(0, nch0)])
            pltpu.sync_copy(col0_hbm.at[sid], colbuf.at[pl.ds(0, nch0)])

        @pl.when(cid == 1)
        def _():
            pltpu.sync_copy(row1_hbm.at[sid], rowbuf.at[pl.ds(0, nch1)])
            pltpu.sync_copy(col1_hbm.at[sid], colbuf.at[pl.ds(0, nch1)])

        nch = jnp.where(cid == 0, nch0, nch1)
        zero32 = jnp.zeros((32,), jnp.bfloat16)

        def zrow(r, carry):
            for j in range(width // 32):
                rows_a[r, pl.ds(j * 32, 32)] = zero32
            return carry

        lax.fori_loop(0, chunk, zrow, 0)
        for k in range(rblk):
            pltpu.sync_copy(rows_a, acc.at[pl.ds(sid * RPT + k * chunk, chunk)])
        plsc.subcore_barrier()

        pltpu.async_copy(g_hbm.at[rowbuf.at[0]], rows_a, sem_a)

        def body(i, carry):
            c0 = 2 * i
            pltpu.make_async_copy(g_hbm.at[rowbuf.at[c0]], rows_a, sem_a).wait()
            pltpu.async_copy(g_hbm.at[rowbuf.at[c0 + 1]], rows_b, sem_b)
            pltpu.sync_copy(rows_a, acc.at[colbuf.at[c0]], add=True)
            pltpu.make_async_copy(g_hbm.at[rowbuf.at[c0 + 1]], rows_b,
                                  sem_b).wait()

            @pl.when(c0 + 2 < nch)
            def _():
                pltpu.async_copy(g_hbm.at[rowbuf.at[c0 + 2]], rows_a, sem_a)

            pltpu.sync_copy(rows_b, acc.at[colbuf.at[c0 + 1]], add=True)
            return carry

        lax.fori_loop(0, nch // 2, body, 0)
        plsc.subcore_barrier()
        for k in range(rblk):
            pltpu.sync_copy(acc.at[pl.ds(sid * RPT + k * chunk, chunk)], rows_a)
            pltpu.sync_copy(rows_a, out_hbm.at[cid,
                                               pl.ds(sid * RPT + k * chunk,
                                                     chunk)])

    return agg


NCH0_H, NCH1_H = 56, 102
NCH0_C, NCH1_C = 62, 96
_sc_agg_h = _make_sc_agg(H, 128, NCH0_H, NCH1_H)
_sc_agg_c = _make_sc_agg(C, 128, NCH0_C, NCH1_C)


def _dense1_body(x_ref, w1_ref, deg_ref, g1_ref, g1b_ref, dis_ref):
    deg = deg_ref[:N] + deg_ref[NPAD:NPAD + N] + 1.0
    dis = lax.rsqrt(deg)[:, None]
    g1 = dis * jnp.dot(x_ref[...], w1_ref[...],
                       preferred_element_type=jnp.float32)
    g1_ref[...] = g1
    g1b_ref[...] = g1.astype(jnp.bfloat16)
    dis_ref[...] = dis


_dense1 = pl.pallas_call(
    _dense1_body,
    out_shape=(jax.ShapeDtypeStruct((N, H), jnp.float32),
               jax.ShapeDtypeStruct((N, H), jnp.bfloat16),
               jax.ShapeDtypeStruct((N, 1), jnp.float32)),
)


def _dense2_body(agg_ref, g1_ref, dis_ref, b1_ref, w2_ref, g2_ref, g2b_ref):
    s = (agg_ref[0, :N, :].astype(jnp.float32)
         + agg_ref[1, :N, :].astype(jnp.float32) + g1_ref[...])
    dis = dis_ref[...]
    h1 = jnp.maximum(dis * s + b1_ref[...][None, :], 0.0)
    g2 = dis * jnp.dot(h1, w2_ref[...], preferred_element_type=jnp.float32)
    g2_ref[...] = g2
    g2b_ref[...] = g2.astype(jnp.bfloat16)


_dense2 = pl.pallas_call(
    _dense2_body,
    out_shape=(jax.ShapeDtypeStruct((N, C), jnp.float32),
               jax.ShapeDtypeStruct((N, C), jnp.bfloat16)),
)


def _dense3_body(agg_ref, g2_ref, dis_ref, b2_ref, out_ref):
    t = (dis_ref[...] * (agg_ref[0, :N, :].astype(jnp.float32)
                         + agg_ref[1, :N, :].astype(jnp.float32)
                         + g2_ref[...])
         + b2_ref[...][None, :])
    m = jnp.max(t, axis=1, keepdims=True)
    lse = m + jnp.log(jnp.sum(jnp.exp(t - m), axis=1, keepdims=True))
    out_ref[...] = t - lse


_dense3 = pl.pallas_call(
    _dense3_body,
    out_shape=jax.ShapeDtypeStruct((N, C), jnp.float32),
)


def kernel(x, edge_index, W1, b1, W2, b2):
    padn = E_PAD - E
    rowp = jnp.concatenate([edge_index[0], jnp.full((padn,), N, jnp.int32)])
    colp = jnp.concatenate([edge_index[1], jnp.full((padn,), N, jnp.int32)])
    row3 = rowp.reshape(NW, NCHUNKS, CHUNK)
    col3 = colp.reshape(NW, NCHUNKS, CHUNK)

    def _split(arr, nch0, nch1):
        cut = NS * nch0 * CHUNK
        return (arr[:cut].reshape(NS, nch0, CHUNK),
                arr[cut:].reshape(NS, nch1, CHUNK))

    r0h, r1h = _split(rowp, NCH0_H, NCH1_H)
    c0h, c1h = _split(colp, NCH0_H, NCH1_H)
    r0c, r1c = _split(rowp, NCH0_C, NCH1_C)
    c0c, c1c = _split(colp, NCH0_C, NCH1_C)

    degp = _sc_degree(col3).reshape(NC * NPAD)
    g1, g1b, dis = _dense1(x, W1, degp)
    g1p = jnp.pad(g1b, ((0, NPAD - N), (0, 0)))
    agg1 = _sc_agg_h(g1p, r0h, c0h, r1h, c1h)
    g2, g2b = _dense2(agg1, g1, dis, b1, W2)
    g2p = jnp.pad(g2b, ((0, NPAD - N), (0, 0)))
    agg2 = _sc_agg_c(g2p, r0c, c0c, r1c, c1c)
    return _dense3(agg2, g2, dis, b2)


# speedup vs baseline: 1.6082x; 1.6082x over previous
import functools

import jax
import jax.numpy as jnp
from jax import lax
from jax.experimental import pallas as pl
from jax.experimental.pallas import tpu as pltpu
from jax.experimental.pallas import tpu_sc as plsc

N = 10000
E = 320000
D = 128
H = 128
C = 64

NC = 2
NS = 16
NW = NC * NS

CHUNK = 128
NCHUNKS = 79
EPW = NCHUNKS * CHUNK
E_PAD = NW * EPW
NPAD = 10240
RPT = NPAD // NS
RBLK = RPT // CHUNK
NROW = NPAD // CHUNK

_MESH = plsc.VectorSubcoreMesh(core_axis_name="c", subcore_axis_name="s")


@functools.partial(
    pl.kernel,
    out_type=jax.ShapeDtypeStruct((NC, NROW, CHUNK), jnp.float32),
    mesh=_MESH,
    scratch_types=[
        pltpu.VMEM((NCHUNKS, CHUNK), jnp.int32),
        pltpu.VMEM((NROW, CHUNK), jnp.float32),
        pltpu.VMEM((NROW // NS, CHUNK), jnp.float32),
        pltpu.VMEM((NROW,), jnp.int32),
        pltpu.VMEM_SHARED((NROW, CHUNK), jnp.float32),
    ],
    compiler_params=pltpu.CompilerParams(use_tc_tiling_on_sc=False,
                                         needs_layout_passes=False),
)
def _sc_degree(col_hbm, out_hbm, colbuf, hist, bounce, idbuf, acc):
    cid = lax.axis_index("c")
    sid = lax.axis_index("s")
    wid = sid * NC + cid
    pltpu.sync_copy(col_hbm.at[wid], colbuf)

    zero16 = jnp.zeros((16,), jnp.float32)
    one16 = jnp.ones((16,), jnp.float32)
    for r in range(NROW // NS):
        for j in range(CHUNK // 16):
            bounce[r, pl.ds(j * 16, 16)] = zero16
    for k in range(NROW // 16):
        idbuf[pl.ds(k * 16, 16)] = lax.iota(jnp.int32, 16) + (k * 16)

    def zrow(r, carry):
        for j in range(CHUNK // 16):
            hist[r, pl.ds(j * 16, 16)] = zero16
        return carry

    lax.fori_loop(0, NROW, zrow, 0)
    pltpu.sync_copy(bounce, acc.at[pl.ds(sid * (NROW // NS), NROW // NS)])

    def body(c, carry):
        for j in range(CHUNK // 16):
            idx = colbuf[c, pl.ds(j * 16, 16)]
            plsc.addupdate_scatter(
                hist, [lax.shift_right_logical(idx, 7),
                       lax.bitwise_and(idx, 127)], one16)
        return carry

    lax.fori_loop(0, NCHUNKS, body, 0)
    plsc.subcore_barrier()
    pltpu.sync_copy(hist, acc.at[idbuf], add=True)
    plsc.subcore_barrier()
    pltpu.sync_copy(acc.at[pl.ds(sid * (NROW // NS), NROW // NS)], bounce)
    pltpu.sync_copy(bounce,
                    out_hbm.at[cid, pl.ds(sid * (NROW // NS), NROW // NS)])


def _make_sc_agg(width, chunk):
    nchunks = EPW // chunk
    npairs = nchunks // 2
    rblk = RPT // chunk

    @functools.partial(
        pl.kernel,
        out_type=jax.ShapeDtypeStruct((NC, NPAD, width), jnp.bfloat16),
        mesh=_MESH,
        scratch_types=[
            pltpu.VMEM((nchunks, chunk), jnp.int32),
            pltpu.VMEM((nchunks, chunk), jnp.int32),
            pltpu.VMEM((chunk, width), jnp.bfloat16),
            pltpu.VMEM((chunk, width), jnp.bfloat16),
            pltpu.VMEM_SHARED((NPAD, width), jnp.bfloat16),
            pltpu.VMEM_SHARED((NPAD, width), jnp.bfloat16),
            pltpu.SemaphoreType.DMA,
            pltpu.SemaphoreType.DMA,
        ],
        compiler_params=pltpu.CompilerParams(use_tc_tiling_on_sc=False),
    )
    def agg(g_hbm, row_hbm, col_hbm, out_hbm,
            rowbuf, colbuf, rows_a, rows_b, table, acc, sem_a, sem_b):
        cid = lax.axis_index("c")
        sid = lax.axis_index("s")
        wid = sid * NC + cid
        pltpu.sync_copy(row_hbm.at[wid], rowbuf)
        pltpu.sync_copy(col_hbm.at[wid], colbuf)

        for k in range(rblk):
            pltpu.sync_copy(g_hbm.at[pl.ds(sid * RPT + k * chunk, chunk)],
                            rows_a)
            pltpu.sync_copy(rows_a, table.at[pl.ds(sid * RPT + k * chunk,
                                                   chunk)])

        zero32 = jnp.zeros((32,), jnp.bfloat16)

        def zrow(r, carry):
            for j in range(width // 32):
                rows_a[r, pl.ds(j * 32, 32)] = zero32
            return carry

        lax.fori_loop(0, chunk, zrow, 0)
        for k in range(rblk):
            pltpu.sync_copy(rows_a, acc.at[pl.ds(sid * RPT + k * chunk, chunk)])
        plsc.subcore_barrier()

        pltpu.async_copy(table.at[rowbuf.at[0]], rows_a, sem_a)

        def body(i, carry):
            c0 = 2 * i
            pltpu.make_async_copy(table.at[rowbuf.at[c0]], rows_a,
                                  sem_a).wait()
            pltpu.async_copy(table.at[rowbuf.at[c0 + 1]], rows_b, sem_b)
            pltpu.sync_copy(rows_a, acc.at[colbuf.at[c0]], add=True)
            pltpu.make_async_copy(table.at[rowbuf.at[c0 + 1]], rows_b,
                                  sem_b).wait()

            @pl.when(c0 + 2 < nchunks)
            def _():
                pltpu.async_copy(table.at[rowbuf.at[c0 + 2]], rows_a, sem_a)

            pltpu.sync_copy(rows_b, acc.at[colbuf.at[c0 + 1]], add=True)
            return carry

        lax.fori_loop(0, npairs, body, 0)
        if nchunks % 2:
            pltpu.make_async_copy(table.at[rowbuf.at[nchunks - 1]], rows_a,
                                  sem_a).wait()
            pltpu.sync_copy(rows_a, acc.at[colbuf.at[nchunks - 1]], add=True)
        plsc.subcore_barrier()
        for k in range(rblk):
            pltpu.sync_copy(acc.at[pl.ds(sid * RPT + k * chunk, chunk)], rows_a)
            pltpu.sync_copy(rows_a, out_hbm.at[cid,
                                               pl.ds(sid * RPT + k * chunk,
                                                     chunk)])

    return agg


_sc_agg_h = _make_sc_agg(H, 128)
_sc_agg_c = _make_sc_agg(C, 128)


def _dense1_body(x_ref, w1_ref, deg_ref, g1_ref, g1b_ref, dis_ref):
    deg = deg_ref[:N] + deg_ref[NPAD:NPAD + N] + 1.0
    dis = lax.rsqrt(deg)[:, None]
    g1 = dis * jnp.dot(x_ref[...], w1_ref[...],
                       preferred_element_type=jnp.float32)
    g1_ref[...] = g1
    g1b_ref[...] = g1.astype(jnp.bfloat16)
    dis_ref[...] = dis


_dense1 = pl.pallas_call(
    _dense1_body,
    out_shape=(jax.ShapeDtypeStruct((N, H), jnp.float32),
               jax.ShapeDtypeStruct((N, H), jnp.bfloat16),
               jax.ShapeDtypeStruct((N, 1), jnp.float32)),
)


def _dense2_body(agg_ref, g1_ref, dis_ref, b1_ref, w2_ref, g2_ref, g2b_ref):
    s = (agg_ref[0, :N, :].astype(jnp.float32)
         + agg_ref[1, :N, :].astype(jnp.float32) + g1_ref[...])
    dis = dis_ref[...]
    h1 = jnp.maximum(dis * s + b1_ref[...][None, :], 0.0)
    g2 = dis * jnp.dot(h1, w2_ref[...], preferred_element_type=jnp.float32)
    g2_ref[...] = g2
    g2b_ref[...] = g2.astype(jnp.bfloat16)


_dense2 = pl.pallas_call(
    _dense2_body,
    out_shape=(jax.ShapeDtypeStruct((N, C), jnp.float32),
               jax.ShapeDtypeStruct((N, C), jnp.bfloat16)),
)


def _dense3_body(agg_ref, g2_ref, dis_ref, b2_ref, out_ref):
    t = (dis_ref[...] * (agg_ref[0, :N, :].astype(jnp.float32)
                         + agg_ref[1, :N, :].astype(jnp.float32)
                         + g2_ref[...])
         + b2_ref[...][None, :])
    m = jnp.max(t, axis=1, keepdims=True)
    lse = m + jnp.log(jnp.sum(jnp.exp(t - m), axis=1, keepdims=True))
    out_ref[...] = t - lse


_dense3 = pl.pallas_call(
    _dense3_body,
    out_shape=jax.ShapeDtypeStruct((N, C), jnp.float32),
)


def kernel(x, edge_index, W1, b1, W2, b2):
    padn = E_PAD - E
    rowp = jnp.concatenate([edge_index[0], jnp.full((padn,), N, jnp.int32)])
    colp = jnp.concatenate([edge_index[1], jnp.full((padn,), N, jnp.int32)])
    row3 = rowp.reshape(NW, NCHUNKS, CHUNK)
    col3 = colp.reshape(NW, NCHUNKS, CHUNK)

    degp = _sc_degree(col3).reshape(NC * NPAD)
    g1, g1b, dis = _dense1(x, W1, degp)
    g1p = jnp.pad(g1b, ((0, NPAD - N), (0, 0)))
    agg1 = _sc_agg_h(g1p, row3, col3)
    g2, g2b = _dense2(agg1, g1, dis, b1, W2)
    g2p = jnp.pad(g2b, ((0, NPAD - N), (0, 0)))
    agg2 = _sc_agg_c(g2p, row3, col3)
    return _dense3(agg2, g2, dis, b2)
